# baseline (speedup 1.0000x reference)
import functools

import jax
import jax.numpy as jnp
from jax import lax
from jax.experimental import pallas as pl

N = 4096
NG = 4
IN_DIM = 16
D = 64
S_DIM = 8
P_DIM = 32
K_NN = 16
N_BLOCKS = 4
ROWS = 256

_f32 = jnp.float32


def _dot(a, b):
    return jnp.dot(a, b, preferred_element_type=_f32)


def _elu(x):
    return jnp.where(x > 0, x, jnp.exp(x) - 1.0)


def _prologue_body(x_ref, b_ref, o_ref):
    x = x_ref[...]
    b = b_ref[...]
    gsel = jnp.zeros((N, IN_DIM), _f32)
    for g in range(NG):
        m = (b == g).astype(_f32)
        cnt = jnp.sum(m)
        sm = jnp.sum(m * x, axis=0, keepdims=True)
        gmean = sm / jnp.maximum(cnt, 1.0)
        gsel = gsel + m * gmean
    o_ref[...] = jnp.concatenate([x, gsel], axis=1)


def _prologue(x, batch_col):
    return pl.pallas_call(
        _prologue_body,
        out_shape=jax.ShapeDtypeStruct((N, 2 * IN_DIM), _f32),
    )(x, batch_col)


def _dense_body(x_ref, W0, b0, W1, b1, W2, b2, Ws, bs, Wh, bh,
                xt_o, s_o, h_o):
    x = x_ref[...]
    x = _elu(_dot(x, W0[...]) + b0[...])
    x = _elu(_dot(x, W1[...]) + b1[...])
    x = jnp.tanh(_dot(x, W2[...]) + b2[...])
    xt_o[...] = x
    s_o[...] = _dot(x, Ws[...]) + bs[...]
    h_o[...] = _dot(x, Wh[...]) + bh[...]


def _dense(xin, p, i):
    fin = xin.shape[1]
    args = [xin]
    for nm in ('W0', 'b0', 'W1', 'b1', 'W2', 'b2', 'Ws', 'bs', 'Wh', 'bh'):
        v = p['b%d_%s' % (i, nm)]
        args.append(v[None, :] if v.ndim == 1 else v)
    return pl.pallas_call(
        _dense_body,
        out_shape=(
            jax.ShapeDtypeStruct((N, D), _f32),
            jax.ShapeDtypeStruct((N, S_DIM), _f32),
            jax.ShapeDtypeStruct((N, P_DIM), _f32),
        ),
    )(*args)


def _knn_body(s_ref, sT_ref, h_ref, bc_ref, br_ref, xt_ref,
              Wo1_ref, Wo2_ref, bo2_ref, y_ref):
    s_r = s_ref[...]
    s_T = sT_ref[...]
    h = h_ref[...]
    bc = bc_ref[...]
    br = br_ref[...]

    sq_r = jnp.sum(s_r * s_r, axis=1, keepdims=True)
    sq_c = jnp.sum(s_T * s_T, axis=0, keepdims=True)
    d = sq_r + sq_c - 2.0 * _dot(s_r, s_T)
    d = jnp.where(bc != br, _f32(1e18), d)

    HR = ROWS // 2
    col = lax.broadcasted_iota(jnp.int32, (HR, N), 1).astype(_f32)
    halves = []
    for dh in (d[:HR], d[HR:]):
        acc_mean = jnp.zeros((HR, P_DIM), _f32)
        acc_max = jnp.full((HR, P_DIM), -jnp.inf, _f32)
        for _ in range(K_NN):
            mv = jnp.min(dh, axis=1, keepdims=True)
            idxv = jnp.min(jnp.where(dh == mv, col, _f32(N)), axis=1,
                           keepdims=True)
            onehot = (col == idxv)
            g = _dot(onehot.astype(_f32), h)
            m = jnp.exp(-10.0 * mv) * g
            acc_mean = acc_mean + m
            acc_max = jnp.maximum(acc_max, m)
            dh = jnp.where(onehot, _f32(jnp.inf), dh)
        halves.append(
            jnp.concatenate([acc_mean * _f32(1.0 / K_NN), acc_max], axis=1))
    agg = jnp.concatenate(halves, axis=0)
    y_ref[...] = (_dot(xt_ref[...], Wo1_ref[...])
                  + _dot(agg, Wo2_ref[...]) + bo2_ref[...])


def _knn_block(s, s_T, h, batch_col, batch_row, x_t, Wo1, Wo2, bo2):
    grid = (N // ROWS,)
    return pl.pallas_call(
        _knn_body,
        grid=grid,
        in_specs=[
            pl.BlockSpec((ROWS, S_DIM), lambda i: (i, 0)),
            pl.BlockSpec((S_DIM, N), lambda i: (0, 0)),
            pl.BlockSpec((N, P_DIM), lambda i: (0, 0)),
            pl.BlockSpec((ROWS, 1), lambda i: (i, 0)),
            pl.BlockSpec((1, N), lambda i: (0, 0)),
            pl.BlockSpec((ROWS, D), lambda i: (i, 0)),
            pl.BlockSpec((D, D), lambda i: (0, 0)),
            pl.BlockSpec((D, D), lambda i: (0, 0)),
            pl.BlockSpec((1, D), lambda i: (0, 0)),
        ],
        out_specs=pl.BlockSpec((ROWS, D), lambda i: (i, 0)),
        out_shape=jax.ShapeDtypeStruct((N, D), _f32),
    )(s, s_T, h, batch_col, batch_row, x_t, Wo1, Wo2, bo2)


def _bn_body(y_ref, g_ref, b_ref, o_ref):
    y = y_ref[...]
    mu = jnp.mean(y, axis=0, keepdims=True)
    var = jnp.mean((y - mu) ** 2, axis=0, keepdims=True)
    o_ref[...] = g_ref[...] * (y - mu) / jnp.sqrt(var + 1e-5) + b_ref[...]


def _bn(y, gamma, beta):
    return pl.pallas_call(
        _bn_body,
        out_shape=jax.ShapeDtypeStruct((N, D), _f32),
    )(y, gamma[None, :], beta[None, :])


def _final_body(f0, f1, f2, f3, W1, b1, W2, b2, W3, b3, o_ref):
    z = jnp.concatenate([f0[...], f1[...], f2[...], f3[...]], axis=1)
    z = jnp.maximum(_dot(z, W1[...]) + b1[...], 0.0)
    z = jnp.maximum(_dot(z, W2[...]) + b2[...], 0.0)
    z = _dot(z, W3[...]) + b3[...]
    z = z - jnp.max(z, axis=1, keepdims=True)
    e = jnp.exp(z)
    o_ref[...] = e / jnp.sum(e, axis=1, keepdims=True)


def _final(feats, p):
    nout = p['f3_W'].shape[1]
    return pl.pallas_call(
        _final_body,
        out_shape=jax.ShapeDtypeStruct((N, nout), _f32),
    )(*feats, p['f1_W'], p['f1_b'][None, :], p['f2_W'], p['f2_b'][None, :],
      p['f3_W'], p['f3_b'][None, :])


def kernel(x, edge_index, batch, num_graphs, params):
    del edge_index, num_graphs
    batch_col = batch[:, None]
    batch_row = batch[None, :]
    xcur = _prologue(x, batch_col)
    feats = []
    for i in range(N_BLOCKS):
        x_t, s, h = _dense(xcur, params, i)
        s_T = s.T
        y = _knn_block(s, s_T, h, batch_col, batch_row, x_t,
                       params['b%d_Wo1' % i], params['b%d_Wo2' % i],
                       params['b%d_bo2' % i][None, :])
        xcur = _bn(y, params['b%d_gamma' % i], params['b%d_beta' % i])
        feats.append(xcur)
    return _final(feats, params)


# speedup vs baseline: 1.2041x; 1.2041x over previous
import functools

import jax
import jax.numpy as jnp
from jax import lax
from jax.experimental import pallas as pl

N = 4096
NG = 4
IN_DIM = 16
D = 64
S_DIM = 8
P_DIM = 32
K_NN = 16
N_BLOCKS = 4
ROWS = 256

_f32 = jnp.float32


def _dot(a, b):
    return jnp.dot(a, b, preferred_element_type=_f32)


def _elu(x):
    return jnp.where(x > 0, x, jnp.exp(x) - 1.0)


def _prologue_body(x_ref, b_ref, o_ref):
    x = x_ref[...]
    b = b_ref[...]
    gsel = jnp.zeros((N, IN_DIM), _f32)
    for g in range(NG):
        m = (b == g).astype(_f32)
        cnt = jnp.sum(m)
        sm = jnp.sum(m * x, axis=0, keepdims=True)
        gmean = sm / jnp.maximum(cnt, 1.0)
        gsel = gsel + m * gmean
    o_ref[...] = jnp.concatenate([x, gsel], axis=1)


def _prologue(x, batch_col):
    return pl.pallas_call(
        _prologue_body,
        out_shape=jax.ShapeDtypeStruct((N, 2 * IN_DIM), _f32),
    )(x, batch_col)


def _dense_body(x_ref, W0, b0, W1, b1, W2, b2, Ws, bs, Wh, bh,
                xt_o, s_o, h_o):
    x = x_ref[...]
    x = _elu(_dot(x, W0[...]) + b0[...])
    x = _elu(_dot(x, W1[...]) + b1[...])
    x = jnp.tanh(_dot(x, W2[...]) + b2[...])
    xt_o[...] = x
    s_o[...] = _dot(x, Ws[...]) + bs[...]
    h_o[...] = _dot(x, Wh[...]) + bh[...]


def _dense(xin, p, i):
    fin = xin.shape[1]
    args = [xin]
    for nm in ('W0', 'b0', 'W1', 'b1', 'W2', 'b2', 'Ws', 'bs', 'Wh', 'bh'):
        v = p['b%d_%s' % (i, nm)]
        args.append(v[None, :] if v.ndim == 1 else v)
    return pl.pallas_call(
        _dense_body,
        out_shape=(
            jax.ShapeDtypeStruct((N, D), _f32),
            jax.ShapeDtypeStruct((N, S_DIM), _f32),
            jax.ShapeDtypeStruct((N, P_DIM), _f32),
        ),
    )(*args)


def _knn_body(s_ref, sT_ref, h_ref, bc_ref, br_ref, xt_ref,
              Wo1_ref, Wo2_ref, bo2_ref, y_ref):
    s_r = s_ref[...]
    s_T = sT_ref[...]
    h = h_ref[...]
    bc = bc_ref[...]
    br = br_ref[...]

    sq_r = jnp.sum(s_r * s_r, axis=1, keepdims=True)
    sq_c = jnp.sum(s_T * s_T, axis=0, keepdims=True)
    d = sq_r + sq_c - 2.0 * _dot(s_r, s_T)
    d = jnp.where(bc != br, _f32(1e18), d)

    col = lax.broadcasted_iota(jnp.int32, (ROWS, N), 1).astype(_f32)
    acc_mean = jnp.zeros((ROWS, P_DIM), _f32)
    acc_max = jnp.full((ROWS, P_DIM), -jnp.inf, _f32)

    for _ in range(K_NN):
        mv = jnp.min(d, axis=1, keepdims=True)
        idxv = jnp.min(jnp.where(d == mv, col, _f32(N)), axis=1,
                       keepdims=True)
        onehot = (col == idxv)
        g = _dot(onehot.astype(_f32), h)
        m = jnp.exp(-10.0 * mv) * g
        acc_mean = acc_mean + m
        acc_max = jnp.maximum(acc_max, m)
        d = jnp.where(onehot, _f32(jnp.inf), d)

    agg = jnp.concatenate([acc_mean * _f32(1.0 / K_NN), acc_max], axis=1)
    y_ref[...] = (_dot(xt_ref[...], Wo1_ref[...])
                  + _dot(agg, Wo2_ref[...]) + bo2_ref[...])


def _knn_block(s, s_T, h, batch_col, batch_row, x_t, Wo1, Wo2, bo2):
    grid = (N // ROWS,)
    return pl.pallas_call(
        _knn_body,
        grid=grid,
        in_specs=[
            pl.BlockSpec((ROWS, S_DIM), lambda i: (i, 0)),
            pl.BlockSpec((S_DIM, N), lambda i: (0, 0)),
            pl.BlockSpec((N, P_DIM), lambda i: (0, 0)),
            pl.BlockSpec((ROWS, 1), lambda i: (i, 0)),
            pl.BlockSpec((1, N), lambda i: (0, 0)),
            pl.BlockSpec((ROWS, D), lambda i: (i, 0)),
            pl.BlockSpec((D, D), lambda i: (0, 0)),
            pl.BlockSpec((D, D), lambda i: (0, 0)),
            pl.BlockSpec((1, D), lambda i: (0, 0)),
        ],
        out_specs=pl.BlockSpec((ROWS, D), lambda i: (i, 0)),
        out_shape=jax.ShapeDtypeStruct((N, D), _f32),
    )(s, s_T, h, batch_col, batch_row, x_t, Wo1, Wo2, bo2)


def _bn_body(y_ref, g_ref, b_ref, o_ref):
    y = y_ref[...]
    mu = jnp.mean(y, axis=0, keepdims=True)
    var = jnp.mean((y - mu) ** 2, axis=0, keepdims=True)
    o_ref[...] = g_ref[...] * (y - mu) / jnp.sqrt(var + 1e-5) + b_ref[...]


def _bn(y, gamma, beta):
    return pl.pallas_call(
        _bn_body,
        out_shape=jax.ShapeDtypeStruct((N, D), _f32),
    )(y, gamma[None, :], beta[None, :])


def _final_body(f0, f1, f2, f3, W1, b1, W2, b2, W3, b3, o_ref):
    z = jnp.concatenate([f0[...], f1[...], f2[...], f3[...]], axis=1)
    z = jnp.maximum(_dot(z, W1[...]) + b1[...], 0.0)
    z = jnp.maximum(_dot(z, W2[...]) + b2[...], 0.0)
    z = _dot(z, W3[...]) + b3[...]
    z = z - jnp.max(z, axis=1, keepdims=True)
    e = jnp.exp(z)
    o_ref[...] = e / jnp.sum(e, axis=1, keepdims=True)


def _final(feats, p):
    nout = p['f3_W'].shape[1]
    return pl.pallas_call(
        _final_body,
        out_shape=jax.ShapeDtypeStruct((N, nout), _f32),
    )(*feats, p['f1_W'], p['f1_b'][None, :], p['f2_W'], p['f2_b'][None, :],
      p['f3_W'], p['f3_b'][None, :])


def kernel(x, edge_index, batch, num_graphs, params):
    del edge_index, num_graphs
    batch_col = batch[:, None]
    batch_row = batch[None, :]
    xcur = _prologue(x, batch_col)
    feats = []
    for i in range(N_BLOCKS):
        x_t, s, h = _dense(xcur, params, i)
        s_T = s.T
        y = _knn_block(s, s_T, h, batch_col, batch_row, x_t,
                       params['b%d_Wo1' % i], params['b%d_Wo2' % i],
                       params['b%d_bo2' % i][None, :])
        xcur = _bn(y, params['b%d_gamma' % i], params['b%d_beta' % i])
        feats.append(xcur)
    return _final(feats, params)
